# Initial kernel scaffold; baseline (speedup 1.0000x reference)
import jax
import jax.numpy as jnp
from jax.experimental import pallas as pl


def kernel(node_feature, edge_index, edge_feature, W):
    raise NotImplementedError("write your pallas kernel here")


# speedup vs baseline: 2.3968x; 2.3968x over previous
import functools

import jax
import jax.numpy as jnp
from jax import lax
from jax.experimental import pallas as pl
from jax.experimental.pallas import tpu as pltpu
from jax.experimental.pallas import tpu_sc as plsc

N_NODES = 10000
N_EDGES = 320000
D_FEAT = 128
D_EDGE = 16
D_OUT = 128

NUM_CORES = 2
NUM_TILES = 16
NUM_WORKERS = NUM_CORES * NUM_TILES
EDGES_PER_TILE = N_EDGES // NUM_WORKERS
BATCH = 80
BATCHES_PER_TILE = EDGES_PER_TILE // BATCH
ACC_ROWS = 10240
ROWS_PER_TILE = ACC_ROWS // NUM_TILES
N_X_CHUNKS = ROWS_PER_TILE // BATCH
E_ROWS = ACC_ROWS // 8
E_ROWS_PER_TILE = E_ROWS // NUM_TILES
N_IO_CHUNKS = N_X_CHUNKS + 1

_mesh = plsc.VectorSubcoreMesh(core_axis_name="c", subcore_axis_name="s")


@functools.partial(
    pl.kernel,
    out_type=(
        jax.ShapeDtypeStruct((NUM_CORES, ACC_ROWS, D_FEAT), jnp.float32),
        jax.ShapeDtypeStruct((NUM_CORES, E_ROWS, D_FEAT), jnp.float32),
    ),
    mesh=_mesh,
    scratch_types=[
        pltpu.VMEM((BATCH,), jnp.int32),
        pltpu.VMEM((BATCH,), jnp.int32),
        pltpu.VMEM((BATCH,), jnp.int32),
        pltpu.VMEM((BATCH, D_FEAT), jnp.float32),
        pltpu.VMEM((BATCH, D_FEAT), jnp.float32),
        pltpu.VMEM_SHARED((ACC_ROWS, D_FEAT), jnp.float32),
        pltpu.VMEM_SHARED((E_ROWS, D_FEAT), jnp.float32),
        pltpu.SemaphoreType.DMA,
    ],
)
def _sc_accumulate(node_hbm, idx_hbm, es_hbm, iota_hbm, iotae_hbm, zx_hbm,
                   px_hbm, pe_hbm, srcb, dstb, d8b, rows, esrows,
                   accx, acce, sem):
    c = lax.axis_index("c")
    s = lax.axis_index("s")
    wid = c * NUM_TILES + s

    pltpu.sync_copy(zx_hbm, rows)
    for k in range(N_X_CHUNKS):
        pltpu.sync_copy(iota_hbm.at[s, k], srcb)
        pltpu.sync_copy(rows, accx.at[srcb])
    pltpu.sync_copy(iotae_hbm.at[s, 0], srcb)
    pltpu.sync_copy(rows, acce.at[srcb])

    plsc.subcore_barrier()

    ebase = wid * EDGES_PER_TILE

    def batch_body(j, carry):
        pltpu.sync_copy(idx_hbm.at[0, wid, j], srcb)
        pltpu.sync_copy(idx_hbm.at[1, wid, j], dstb)
        pltpu.sync_copy(idx_hbm.at[2, wid, j], d8b)
        pltpu.async_copy(node_hbm.at[srcb], rows, sem).wait()
        pltpu.sync_copy(rows, accx.at[dstb], add=True)
        eoff = pl.multiple_of(ebase + j * BATCH, 16)
        pltpu.sync_copy(es_hbm.at[pl.ds(eoff, BATCH), :], esrows)
        pltpu.sync_copy(esrows, acce.at[d8b], add=True)
        return carry

    lax.fori_loop(0, BATCHES_PER_TILE, batch_body, 0)

    plsc.subcore_barrier()

    r0 = s * ROWS_PER_TILE
    for k in range(N_X_CHUNKS):
        pltpu.sync_copy(iota_hbm.at[s, k], srcb)
        pltpu.async_copy(accx.at[srcb], rows, sem).wait()
        pltpu.sync_copy(rows, px_hbm.at[c, pl.ds(r0 + k * BATCH, BATCH), :])
    pltpu.sync_copy(iotae_hbm.at[s, 0], srcb)
    pltpu.async_copy(acce.at[srcb], esrows, sem).wait()
    pltpu.sync_copy(esrows,
                    pe_hbm.at[c, pl.ds(s * E_ROWS_PER_TILE, E_ROWS_PER_TILE), :])


_ROW_BLK = 1000


def _mm_body(p_ref, f_ref, w1_ref, w2_ref, o_ref):
    p = p_ref[0] + p_ref[1]
    f = f_ref[0] + f_ref[1]
    o_ref[...] = (
        jnp.dot(p, w1_ref[...], preferred_element_type=jnp.float32)
        + jnp.dot(f, w2_ref[...], preferred_element_type=jnp.float32)
    )


def _tc_combine(P, F, W1T, W2T):
    return pl.pallas_call(
        _mm_body,
        grid=(N_NODES // _ROW_BLK,),
        in_specs=[
            pl.BlockSpec((NUM_CORES, _ROW_BLK, D_FEAT), lambda i: (0, i, 0)),
            pl.BlockSpec((NUM_CORES, _ROW_BLK, D_EDGE), lambda i: (0, i, 0)),
            pl.BlockSpec((D_FEAT, D_OUT), lambda i: (0, 0)),
            pl.BlockSpec((D_EDGE, D_OUT), lambda i: (0, 0)),
        ],
        out_specs=pl.BlockSpec((_ROW_BLK, D_OUT), lambda i: (i, 0)),
        out_shape=jax.ShapeDtypeStruct((N_NODES, D_OUT), jnp.float32),
    )(P, F, W1T, W2T)


def kernel(node_feature, edge_index, edge_feature, W):
    src = edge_index[0].astype(jnp.int32)
    dst = edge_index[1].astype(jnp.int32)
    idx = jnp.stack([src, dst, dst // 8])
    idx = idx.reshape(3, NUM_WORKERS, BATCHES_PER_TILE, BATCH)
    slot_oh = (dst % 8)[:, None] == jnp.arange(8, dtype=jnp.int32)[None, :]
    es = (slot_oh.astype(jnp.float32)[:, :, None]
          * edge_feature[:, None, :]).reshape(N_EDGES, D_FEAT)
    iota_x = jnp.arange(ACC_ROWS, dtype=jnp.int32).reshape(
        NUM_TILES, N_X_CHUNKS, BATCH)
    iota_e = jnp.arange(E_ROWS, dtype=jnp.int32).reshape(
        NUM_TILES, 1, E_ROWS_PER_TILE)
    zx = jnp.zeros((BATCH, D_FEAT), jnp.float32)
    P, Pe = _sc_accumulate(node_feature, idx, es, iota_x, iota_e, zx)
    F = Pe.reshape(NUM_CORES, ACC_ROWS, D_EDGE)[:, :N_NODES]
    P = P[:, :N_NODES]
    W1T = W[:, :D_FEAT].T
    W2T = W[:, D_FEAT:].T
    return _tc_combine(P, F, W1T, W2T)


# speedup vs baseline: 3.2433x; 1.3532x over previous
import functools

import jax
import jax.numpy as jnp
from jax import lax
from jax.experimental import pallas as pl
from jax.experimental.pallas import tpu as pltpu
from jax.experimental.pallas import tpu_sc as plsc

N_NODES = 10000
N_EDGES = 320000
D_FEAT = 128
D_EDGE = 16
D_OUT = 128

NUM_CORES = 2
NUM_TILES = 16
NUM_WORKERS = NUM_CORES * NUM_TILES
EDGES_PER_TILE = N_EDGES // NUM_WORKERS
BATCH = 80
BATCHES_PER_TILE = EDGES_PER_TILE // BATCH
ACC_ROWS = 10240
ROWS_PER_TILE = ACC_ROWS // NUM_TILES
N_X_CHUNKS = ROWS_PER_TILE // BATCH
E_ROWS = ACC_ROWS // 8
E_ROWS_PER_TILE = E_ROWS // NUM_TILES
N_IO_CHUNKS = N_X_CHUNKS + 1

_mesh = plsc.VectorSubcoreMesh(core_axis_name="c", subcore_axis_name="s")


@functools.partial(
    pl.kernel,
    out_type=(
        jax.ShapeDtypeStruct((NUM_CORES, ACC_ROWS, D_FEAT), jnp.float32),
        jax.ShapeDtypeStruct((NUM_CORES, E_ROWS, D_FEAT), jnp.float32),
    ),
    mesh=_mesh,
    scratch_types=[
        pltpu.VMEM((3, BATCH), jnp.int32),
        pltpu.VMEM((3, BATCH), jnp.int32),
        pltpu.VMEM((BATCH, D_FEAT), jnp.float32),
        pltpu.VMEM((BATCH, D_FEAT), jnp.float32),
        pltpu.VMEM_SHARED((ACC_ROWS, D_FEAT), jnp.float32),
        pltpu.VMEM_SHARED((E_ROWS, D_FEAT), jnp.float32),
        pltpu.SemaphoreType.DMA,
        pltpu.SemaphoreType.DMA,
    ],
)
def _sc_accumulate(node_hbm, idx_hbm, es_hbm, iota_hbm, iotae_hbm, zx_hbm,
                   px_hbm, pe_hbm, ibA, ibB, rowsA, rowsB,
                   accx, acce, semA, semB):
    c = lax.axis_index("c")
    s = lax.axis_index("s")
    wid = c * NUM_TILES + s

    pltpu.sync_copy(zx_hbm, rowsA)
    for k in range(N_X_CHUNKS):
        pltpu.sync_copy(iota_hbm.at[s, k], ibA.at[0])
        pltpu.sync_copy(rowsA, accx.at[ibA.at[0]])
    pltpu.sync_copy(iotae_hbm.at[s, 0], ibA.at[0])
    pltpu.sync_copy(rowsA, acce.at[ibA.at[0]])

    plsc.subcore_barrier()

    ebase = wid * EDGES_PER_TILE

    def prefetch(j, ib, rows, sem):
        pltpu.sync_copy(idx_hbm.at[wid, j], ib)
        return pltpu.async_copy(node_hbm.at[ib.at[0]], rows, sem)

    def finish(j, ib, rows, sem):
        pltpu.make_async_copy(node_hbm.at[ib.at[0]], rows, sem).wait()
        pltpu.sync_copy(rows, accx.at[ib.at[1]], add=True)
        eoff = pl.multiple_of(ebase + j * BATCH, 16)
        pltpu.sync_copy(es_hbm.at[pl.ds(eoff, BATCH), :], rows)
        pltpu.sync_copy(rows, acce.at[ib.at[2]], add=True)

    prefetch(0, ibA, rowsA, semA)

    def pair_body(t, carry):
        jA = 2 * t
        prefetch(jA + 1, ibB, rowsB, semB)
        finish(jA, ibA, rowsA, semA)
        prefetch(jA + 2, ibA, rowsA, semA)
        finish(jA + 1, ibB, rowsB, semB)
        return carry

    lax.fori_loop(0, (BATCHES_PER_TILE - 1) // 2, pair_body, 0)
    finish(BATCHES_PER_TILE - 1, ibA, rowsA, semA)

    plsc.subcore_barrier()

    r0 = s * ROWS_PER_TILE
    for k in range(N_X_CHUNKS):
        pltpu.sync_copy(iota_hbm.at[s, k], ibA.at[0])
        pltpu.async_copy(accx.at[ibA.at[0]], rowsA, semA).wait()
        pltpu.sync_copy(rowsA, px_hbm.at[c, pl.ds(r0 + k * BATCH, BATCH), :])
    pltpu.sync_copy(iotae_hbm.at[s, 0], ibA.at[0])
    pltpu.async_copy(acce.at[ibA.at[0]], rowsA, semA).wait()
    pltpu.sync_copy(rowsA,
                    pe_hbm.at[c, pl.ds(s * E_ROWS_PER_TILE, E_ROWS_PER_TILE), :])


_ROW_BLK = 1000


def _mm_body(p_ref, f_ref, w1_ref, w2_ref, o_ref):
    p = p_ref[0] + p_ref[1]
    f = f_ref[0] + f_ref[1]
    o_ref[...] = (
        jnp.dot(p, w1_ref[...], preferred_element_type=jnp.float32)
        + jnp.dot(f, w2_ref[...], preferred_element_type=jnp.float32)
    )


def _tc_combine(P, F, W1T, W2T):
    return pl.pallas_call(
        _mm_body,
        grid=(N_NODES // _ROW_BLK,),
        in_specs=[
            pl.BlockSpec((NUM_CORES, _ROW_BLK, D_FEAT), lambda i: (0, i, 0)),
            pl.BlockSpec((NUM_CORES, _ROW_BLK, D_EDGE), lambda i: (0, i, 0)),
            pl.BlockSpec((D_FEAT, D_OUT), lambda i: (0, 0)),
            pl.BlockSpec((D_EDGE, D_OUT), lambda i: (0, 0)),
        ],
        out_specs=pl.BlockSpec((_ROW_BLK, D_OUT), lambda i: (i, 0)),
        out_shape=jax.ShapeDtypeStruct((N_NODES, D_OUT), jnp.float32),
    )(P, F, W1T, W2T)


def kernel(node_feature, edge_index, edge_feature, W):
    src = edge_index[0].astype(jnp.int32)
    dst = edge_index[1].astype(jnp.int32)
    idx = jnp.stack([src, dst, dst // 8])
    idx = idx.reshape(3, NUM_WORKERS, BATCHES_PER_TILE, BATCH)
    idx = idx.transpose(1, 2, 0, 3)
    slot_oh = (dst % 8)[:, None] == jnp.arange(8, dtype=jnp.int32)[None, :]
    es = (slot_oh.astype(jnp.float32)[:, :, None]
          * edge_feature[:, None, :]).reshape(N_EDGES, D_FEAT)
    iota_x = jnp.arange(ACC_ROWS, dtype=jnp.int32).reshape(
        NUM_TILES, N_X_CHUNKS, BATCH)
    iota_e = jnp.arange(E_ROWS, dtype=jnp.int32).reshape(
        NUM_TILES, 1, E_ROWS_PER_TILE)
    zx = jnp.zeros((BATCH, D_FEAT), jnp.float32)
    P, Pe = _sc_accumulate(node_feature, idx, es, iota_x, iota_e, zx)
    F = Pe.reshape(NUM_CORES, ACC_ROWS, D_EDGE)[:, :N_NODES]
    P = P[:, :N_NODES]
    W1T = W[:, :D_FEAT].T
    W2T = W[:, D_FEAT:].T
    return _tc_combine(P, F, W1T, W2T)


# speedup vs baseline: 3.9141x; 1.2068x over previous
import functools

import jax
import jax.numpy as jnp
from jax import lax
from jax.experimental import pallas as pl
from jax.experimental.pallas import tpu as pltpu
from jax.experimental.pallas import tpu_sc as plsc

N_NODES = 10000
N_EDGES = 320000
D_FEAT = 128
D_EDGE = 16
D_OUT = 128

NUM_CORES = 2
NUM_TILES = 16
NUM_WORKERS = NUM_CORES * NUM_TILES
EDGES_PER_TILE = N_EDGES // NUM_WORKERS
BATCH = 80
BATCHES_PER_TILE = EDGES_PER_TILE // BATCH
ACC_ROWS = 10240
ROWS_PER_TILE = ACC_ROWS // NUM_TILES
N_X_CHUNKS = ROWS_PER_TILE // BATCH
E_ROWS = ACC_ROWS // 8
E_ROWS_PER_TILE = E_ROWS // NUM_TILES
N_IO_CHUNKS = N_X_CHUNKS + 1

_mesh = plsc.VectorSubcoreMesh(core_axis_name="c", subcore_axis_name="s")


@functools.partial(
    pl.kernel,
    out_type=(
        jax.ShapeDtypeStruct((NUM_CORES, ACC_ROWS, D_FEAT), jnp.float32),
        jax.ShapeDtypeStruct((NUM_CORES, E_ROWS, D_FEAT), jnp.float32),
    ),
    mesh=_mesh,
    scratch_types=[
        pltpu.VMEM((3, BATCH), jnp.int32),
        pltpu.VMEM((3, BATCH), jnp.int32),
        pltpu.VMEM((BATCH, D_FEAT), jnp.float32),
        pltpu.VMEM((BATCH, D_FEAT), jnp.float32),
        pltpu.VMEM((BATCH, D_FEAT), jnp.float32),
        pltpu.VMEM_SHARED((ACC_ROWS, D_FEAT), jnp.float32),
        pltpu.VMEM_SHARED((E_ROWS, D_FEAT), jnp.float32),
        pltpu.SemaphoreType.DMA,
        pltpu.SemaphoreType.DMA,
        pltpu.SemaphoreType.DMA,
    ],
)
def _sc_accumulate(node_hbm, idx_hbm, es_hbm, iota_hbm, iotae_hbm, zx_hbm,
                   px_hbm, pe_hbm, ibA, ibB, rowsA, rowsB, esb,
                   accx, acce, semA, semB, esem):
    c = lax.axis_index("c")
    s = lax.axis_index("s")
    wid = c * NUM_TILES + s

    pltpu.sync_copy(zx_hbm, rowsA)
    for k in range(N_X_CHUNKS):
        pltpu.sync_copy(iota_hbm.at[s, k], ibA.at[0])
        pltpu.sync_copy(rowsA, accx.at[ibA.at[0]])
    pltpu.sync_copy(iotae_hbm.at[s, 0], ibA.at[0])
    pltpu.sync_copy(rowsA, acce.at[ibA.at[0]])

    plsc.subcore_barrier()

    ebase = wid * EDGES_PER_TILE

    def es_off(j):
        return pl.multiple_of(ebase + j * BATCH, 16)

    def es_issue(j):
        pltpu.async_copy(es_hbm.at[pl.ds(es_off(j), BATCH), :], esb, esem)

    def prefetch(j, ib, rows, sem):
        pltpu.sync_copy(idx_hbm.at[wid, j], ib)
        pltpu.async_copy(node_hbm.at[ib.at[0]], rows, sem)

    def finish(j, ib, rows, sem):
        pltpu.make_async_copy(node_hbm.at[ib.at[0]], rows, sem).wait()
        pltpu.sync_copy(rows, accx.at[ib.at[1]], add=True)
        pltpu.make_async_copy(es_hbm.at[pl.ds(es_off(j), BATCH), :],
                              esb, esem).wait()
        pltpu.sync_copy(esb, acce.at[ib.at[2]], add=True)
        es_issue(lax.min(j + 1, BATCHES_PER_TILE - 1))

    es_issue(0)
    prefetch(0, ibA, rowsA, semA)

    def pair_body(t, carry):
        jA = 2 * t
        prefetch(jA + 1, ibB, rowsB, semB)
        finish(jA, ibA, rowsA, semA)
        prefetch(jA + 2, ibA, rowsA, semA)
        finish(jA + 1, ibB, rowsB, semB)
        return carry

    lax.fori_loop(0, (BATCHES_PER_TILE - 1) // 2, pair_body, 0)
    finish(BATCHES_PER_TILE - 1, ibA, rowsA, semA)
    pltpu.make_async_copy(
        es_hbm.at[pl.ds(es_off(BATCHES_PER_TILE - 1), BATCH), :],
        esb, esem).wait()

    plsc.subcore_barrier()

    r0 = s * ROWS_PER_TILE
    for k in range(N_X_CHUNKS):
        pltpu.sync_copy(iota_hbm.at[s, k], ibA.at[0])
        pltpu.async_copy(accx.at[ibA.at[0]], rowsA, semA).wait()
        pltpu.sync_copy(rowsA, px_hbm.at[c, pl.ds(r0 + k * BATCH, BATCH), :])
    pltpu.sync_copy(iotae_hbm.at[s, 0], ibA.at[0])
    pltpu.async_copy(acce.at[ibA.at[0]], rowsA, semA).wait()
    pltpu.sync_copy(rowsA,
                    pe_hbm.at[c, pl.ds(s * E_ROWS_PER_TILE, E_ROWS_PER_TILE), :])


_ROW_BLK = 1000


def _mm_body(p_ref, f_ref, w1_ref, w2_ref, o_ref):
    p = p_ref[0] + p_ref[1]
    f = f_ref[0] + f_ref[1]
    o_ref[...] = (
        jnp.dot(p, w1_ref[...], preferred_element_type=jnp.float32)
        + jnp.dot(f, w2_ref[...], preferred_element_type=jnp.float32)
    )


def _tc_combine(P, F, W1T, W2T):
    return pl.pallas_call(
        _mm_body,
        grid=(N_NODES // _ROW_BLK,),
        in_specs=[
            pl.BlockSpec((NUM_CORES, _ROW_BLK, D_FEAT), lambda i: (0, i, 0)),
            pl.BlockSpec((NUM_CORES, _ROW_BLK, D_EDGE), lambda i: (0, i, 0)),
            pl.BlockSpec((D_FEAT, D_OUT), lambda i: (0, 0)),
            pl.BlockSpec((D_EDGE, D_OUT), lambda i: (0, 0)),
        ],
        out_specs=pl.BlockSpec((_ROW_BLK, D_OUT), lambda i: (i, 0)),
        out_shape=jax.ShapeDtypeStruct((N_NODES, D_OUT), jnp.float32),
    )(P, F, W1T, W2T)


def kernel(node_feature, edge_index, edge_feature, W):
    src = edge_index[0].astype(jnp.int32)
    dst = edge_index[1].astype(jnp.int32)
    idx = jnp.stack([src, dst, dst // 8])
    idx = idx.reshape(3, NUM_WORKERS, BATCHES_PER_TILE, BATCH)
    idx = idx.transpose(1, 2, 0, 3)
    slot_oh = (dst % 8)[:, None] == jnp.arange(8, dtype=jnp.int32)[None, :]
    es = (slot_oh.astype(jnp.float32)[:, :, None]
          * edge_feature[:, None, :]).reshape(N_EDGES, D_FEAT)
    iota_x = jnp.arange(ACC_ROWS, dtype=jnp.int32).reshape(
        NUM_TILES, N_X_CHUNKS, BATCH)
    iota_e = jnp.arange(E_ROWS, dtype=jnp.int32).reshape(
        NUM_TILES, 1, E_ROWS_PER_TILE)
    zx = jnp.zeros((BATCH, D_FEAT), jnp.float32)
    P, Pe = _sc_accumulate(node_feature, idx, es, iota_x, iota_e, zx)
    F = Pe.reshape(NUM_CORES, ACC_ROWS, D_EDGE)[:, :N_NODES]
    P = P[:, :N_NODES]
    W1T = W[:, :D_FEAT].T
    W2T = W[:, D_FEAT:].T
    return _tc_combine(P, F, W1T, W2T)
